# baseline (speedup 1.0000x reference)
import functools

import jax
import jax.numpy as jnp
from jax import lax
from jax.experimental import pallas as pl
from jax.experimental.pallas import tpu as pltpu
from jax.experimental.pallas import tpu_sc as plsc

N = 10000
E = 160000
D_IN = 128
D_H1 = 4096
D_H2 = 1024
D_OUT = 128

NC = 2
NS = 16
NW = NC * NS

CK = 64
CW = 120
CWC1 = 37
NPAD2 = 10112
NPAD1 = 10240
RPS = NPAD2 // NS
DPS = NPAD1 // NS

def _worker_id():
  c = lax.axis_index("c")
  s = lax.axis_index("s")
  return s * NC + c


def _deg_body(dstb_hbm, zero1_hbm, out_hbm, dst_v, ones_v, acc_sh):
  c = lax.axis_index("c")
  s = lax.axis_index("s")
  w = _worker_id()

  def fill_ones(i, _):
    ones_v[pl.ds(i * 16, 16)] = jnp.ones((16,), jnp.float32)
    return ()
  lax.fori_loop(0, CK // 16, fill_ones, ())

  pltpu.sync_copy(zero1_hbm.at[pl.ds(s * DPS, DPS)],
                  acc_sh.at[pl.ds(s * DPS, DPS)])
  pltpu.sync_copy(dstb_hbm.at[pl.ds(w * CW, CW)], dst_v)
  plsc.subcore_barrier()

  def chunk(g, _):
    pltpu.sync_copy(ones_v, acc_sh.at[dst_v.at[g]], add=True)
    return ()
  lax.fori_loop(0, jnp.where(c == 0, CW, CWC1), chunk, ())

  plsc.subcore_barrier()
  off = pl.multiple_of(c * NPAD1 + s * DPS, 128)
  pltpu.sync_copy(acc_sh.at[pl.ds(s * DPS, DPS)], out_hbm.at[pl.ds(off, DPS)])


NBUF = 4
LA = 1


def _agg_body(table_hbm, pkb_hbm, zero2_hbm, out_hbm,
              pk_v, sidx_v, didx_v, rows_v, acc_sh, *sems, nplanes, out4d):
  gsems, ssems = sems[:NBUF], sems[NBUF:]
  c = lax.axis_index("c")
  s = lax.axis_index("s")
  w = _worker_id()
  c0 = c == 0

  pltpu.sync_copy(pkb_hbm.at[pl.ds(w * CW, CW)], pk_v)

  def run_pass(table_f, dump_dst):
    pltpu.sync_copy(zero2_hbm.at[pl.ds(s * RPS, RPS)],
                    acc_sh.at[pl.ds(s * RPS, RPS)])
    plsc.subcore_barrier()

    def g_start(g, b):
      for j in range(CK // 16):
        v = pk_v[g, pl.ds(j * 16, 16)]
        sidx_v[b, pl.ds(j * 16, 16)] = lax.shift_right_logical(v, 14)
        didx_v[b, pl.ds(j * 16, 16)] = v & 16383
      pltpu.make_async_copy(
          table_f.at[sidx_v.at[b]], rows_v.at[b], gsems[b]).start()

    def g_wait(b):
      pltpu.make_async_copy(
          table_f.at[sidx_v.at[b]], rows_v.at[b], gsems[b]).wait()

    def s_start(b):
      pltpu.make_async_copy(
          rows_v.at[b], acc_sh.at[didx_v.at[b]], ssems[b]).start(add=True)

    def s_wait(b):
      pltpu.make_async_copy(
          rows_v.at[b], acc_sh.at[didx_v.at[b]], ssems[b]).wait()

    for g in range(LA):
      g_start(g, g % NBUF)
    for g in range(CW):
      b = g % NBUF

      def work(b=b):
        g_wait(b)
        s_start(b)
      if g < CWC1:
        work()
      else:
        pl.when(c0)(work)
      la = g + LA
      if la < CW:
        lb = la % NBUF

        def ahead(la=la, lb=lb):
          if la >= NBUF:
            s_wait(lb)
          g_start(la, lb)
        if la < CWC1:
          ahead()
        else:
          pl.when(c0)(ahead)

    def drain0():
      for g in range(CW - NBUF, CW):
        s_wait(g % NBUF)

    def drain1():
      for g in range(CWC1 - NBUF, CWC1):
        s_wait(g % NBUF)
    pl.when(c0)(drain0)
    pl.when(c != 0)(drain1)

    plsc.subcore_barrier()
    pltpu.sync_copy(acc_sh.at[pl.ds(s * RPS, RPS)], dump_dst)

  if not out4d:
    run_pass(table_hbm, out_hbm.at[c, pl.ds(s * RPS, RPS)])
  else:
    def pass_body(f, _):
      run_pass(table_hbm.at[f], out_hbm.at[c, f, pl.ds(s * RPS, RPS)])
      plsc.subcore_barrier()
      return ()
    lax.fori_loop(0, nplanes, pass_body, ())


@functools.cache
def _sc_kernels():
  mesh = plsc.VectorSubcoreMesh(
      core_axis_name="c", subcore_axis_name="s", num_cores=NC, num_subcores=NS)
  agg_scratch = [
      pltpu.VMEM((CW, CK), jnp.int32),
      pltpu.VMEM((NBUF, CK), jnp.int32),
      pltpu.VMEM((NBUF, CK), jnp.int32),
      pltpu.VMEM((NBUF, CK, 128), jnp.float32),
      pltpu.VMEM_SHARED((NPAD2, 128), jnp.float32),
  ] + [pltpu.SemaphoreType.DMA] * (2 * NBUF)

  deg = pl.kernel(
      _deg_body,
      out_type=jax.ShapeDtypeStruct((NC * NPAD1,), jnp.float32),
      mesh=mesh,
      scratch_types=[
          pltpu.VMEM((CW, CK), jnp.int32),
          pltpu.VMEM((CK,), jnp.float32),
          pltpu.VMEM_SHARED((NPAD1,), jnp.float32),
      ],
  )

  def agg128_body(table_hbm, pkb_hbm, zero2_hbm, out_hbm,
                  pk_v, sidx_v, didx_v, rows_v, acc_sh, *sems):
    _agg_body(table_hbm, pkb_hbm, zero2_hbm, out_hbm,
              pk_v, sidx_v, didx_v, rows_v, acc_sh, *sems,
              nplanes=1, out4d=False)

  def agg1024_body(table_hbm, pkb_hbm, zero2_hbm, out_hbm,
                   pk_v, sidx_v, didx_v, rows_v, acc_sh, *sems):
    _agg_body(table_hbm, pkb_hbm, zero2_hbm, out_hbm,
              pk_v, sidx_v, didx_v, rows_v, acc_sh, *sems,
              nplanes=4, out4d=True)

  agg128 = pl.kernel(
      agg128_body,
      out_type=jax.ShapeDtypeStruct((NC, NPAD2, 128), jnp.float32),
      mesh=mesh, scratch_types=agg_scratch)
  agg1024 = pl.kernel(
      agg1024_body,
      out_type=jax.ShapeDtypeStruct((NC, 4, NPAD2, 128), jnp.float32),
      mesh=mesh, scratch_types=agg_scratch)
  return deg, agg128, agg1024


RT = 400
MT = 2000
NGRID = N // RT


def _prep_body(deg_ref, x_ref, dinv_ref, xs_ref):
  ones = jnp.ones((NC, 128), jnp.float32)
  degb = lax.dot_general(deg_ref[...], ones, (((0,), (0,)), ((), ())),
                         preferred_element_type=jnp.float32)
  dinv = lax.rsqrt(degb + 1.0)
  dinv_ref[...] = dinv
  xs_ref[...] = x_ref[...] * dinv


def _tc_prep(deg_parts, x):
  rp = 512
  return pl.pallas_call(
      _prep_body,
      grid=(pl.cdiv(N, rp),),
      in_specs=[
          pl.BlockSpec((NC, rp), lambda r: (0, r)),
          pl.BlockSpec((rp, 128), lambda r: (r, 0)),
      ],
      out_specs=[
          pl.BlockSpec((rp, 128), lambda r: (r, 0)),
          pl.BlockSpec((rp, 128), lambda r: (r, 0)),
      ],
      out_shape=[
          jax.ShapeDtypeStruct((N, 128), jnp.float32),
          jax.ShapeDtypeStruct((N, 128), jnp.float32),
      ],
  )(deg_parts, x)


def _m1_body(t1_ref, xs_ref, dinv_ref, w_ref, b_ref, out_ref):
  u = dinv_ref[...] * (t1_ref[0] + t1_ref[1] + xs_ref[...])
  for n in range(D_H1 // 512):
    a = jnp.dot(u, w_ref[:, n * 512:(n + 1) * 512],
                preferred_element_type=jnp.float32)
    out_ref[:, n * 512:(n + 1) * 512] = jnp.maximum(
        a + b_ref[:, n * 512:(n + 1) * 512], 0.0).astype(jnp.bfloat16)


def _tc_m1(t1, xs, dinv, w1, b1):
  return pl.pallas_call(
      _m1_body,
      grid=(N // MT,),
      in_specs=[
          pl.BlockSpec((NC, MT, 128), lambda r: (0, r, 0)),
          pl.BlockSpec((MT, 128), lambda r: (r, 0)),
          pl.BlockSpec((MT, 128), lambda r: (r, 0)),
          pl.BlockSpec((128, D_H1), lambda r: (0, 0)),
          pl.BlockSpec((1, D_H1), lambda r: (0, 0)),
      ],
      out_specs=pl.BlockSpec((MT, D_H1), lambda r: (r, 0)),
      out_shape=jax.ShapeDtypeStruct((N, D_H1), jnp.bfloat16),
  )(t1, xs, dinv, w1, b1)


def _m2_body(h1_ref, w_ref, dinv_ref, out_ref, acc_ref):
  k = pl.program_id(1)

  @pl.when(k == 0)
  def _():
    acc_ref[...] = jnp.zeros_like(acc_ref)

  acc_ref[...] += jnp.dot(h1_ref[...], w_ref[...],
                          preferred_element_type=jnp.float32)

  @pl.when(k == pl.num_programs(1) - 1)
  def _():
    dinv = dinv_ref[...]
    for t in range(4):
      out_ref[t] = dinv * acc_ref[:, t * 128:(t + 1) * 128]


def _tc_m2(h1, dinv, w2half):
  kk = 512
  return pl.pallas_call(
      _m2_body,
      grid=(N // MT, D_H1 // kk),
      in_specs=[
          pl.BlockSpec((MT, kk), lambda r, k: (r, k)),
          pl.BlockSpec((kk, 512), lambda r, k: (k, 0)),
          pl.BlockSpec((MT, 128), lambda r, k: (r, 0)),
      ],
      out_specs=pl.BlockSpec((4, MT, 128), lambda r, k: (0, r, 0)),
      out_shape=jax.ShapeDtypeStruct((4, N, 128), jnp.float32),
      scratch_shapes=[pltpu.VMEM((MT, 512), jnp.float32)],
      compiler_params=pltpu.CompilerParams(
          dimension_semantics=("parallel", "arbitrary")),
  )(h1, w2half, dinv)


def _m3a_body(t2_ref, gs2_ref, dinv_ref, b2_ref, w3_ref, out_ref, acc_ref):
  dinv = dinv_ref[...]
  acc_ref[...] = jnp.zeros_like(acc_ref)
  for f in range(4):
    h2 = jnp.maximum(
        dinv * (t2_ref[0, f] + t2_ref[1, f] + gs2_ref[f]) + b2_ref[f], 0.0)
    acc_ref[...] += jnp.dot(h2, w3_ref[f * 128:(f + 1) * 128],
                            preferred_element_type=jnp.float32)
  out_ref[...] = acc_ref[...]


def _m3b_body(t2_ref, gs2_ref, dinv_ref, b2_ref, w3_ref, part_ref,
              out_ref, acc_ref):
  dinv = dinv_ref[...]
  acc_ref[...] = part_ref[...]
  for f in range(4):
    h2 = jnp.maximum(
        dinv * (t2_ref[0, f] + t2_ref[1, f] + gs2_ref[f]) + b2_ref[f], 0.0)
    acc_ref[...] += jnp.dot(h2, w3_ref[f * 128:(f + 1) * 128],
                            preferred_element_type=jnp.float32)
  out_ref[...] = dinv * acc_ref[...]


def _m3_specs(extra):
  return dict(
      grid=(N // MT,),
      in_specs=[
          pl.BlockSpec((NC, 4, MT, 128), lambda r: (0, 0, r, 0)),
          pl.BlockSpec((4, MT, 128), lambda r: (0, r, 0)),
          pl.BlockSpec((MT, 128), lambda r: (r, 0)),
          pl.BlockSpec((4, 1, 128), lambda r: (0, 0, 0)),
          pl.BlockSpec((512, 128), lambda r: (0, 0)),
      ] + extra,
      out_specs=pl.BlockSpec((MT, 128), lambda r: (r, 0)),
      out_shape=jax.ShapeDtypeStruct((N, 128), jnp.float32),
      scratch_shapes=[pltpu.VMEM((MT, 128), jnp.float32)],
  )


def _tc_m3a(t2, gs2, dinv, b2, w3):
  return pl.pallas_call(_m3a_body, **_m3_specs([]))(t2, gs2, dinv, b2, w3)


def _tc_m3b(t2, gs2, dinv, b2, w3, part):
  return pl.pallas_call(
      _m3b_body,
      **_m3_specs([pl.BlockSpec((MT, 128), lambda r: (r, 0))]),
  )(t2, gs2, dinv, b2, w3, part)


def _final_body(t3_ref, gs3_ref, dinv_ref, b3_ref, out_ref):
  u = dinv_ref[...] * (t3_ref[0] + t3_ref[1] + gs3_ref[...]) + b3_ref[...]
  m = jnp.max(u, axis=1, keepdims=True)
  e = jnp.exp(u - m)
  lse = jnp.log(jnp.sum(e, axis=1, keepdims=True)) + m
  out_ref[...] = u - lse


def _tc_final(t3, gs3, dinv, b3):
  return pl.pallas_call(
      _final_body,
      grid=(NGRID,),
      in_specs=[
          pl.BlockSpec((NC, RT, 128), lambda r: (0, r, 0)),
          pl.BlockSpec((RT, 128), lambda r: (r, 0)),
          pl.BlockSpec((RT, 128), lambda r: (r, 0)),
          pl.BlockSpec((1, 128), lambda r: (0, 0)),
      ],
      out_specs=pl.BlockSpec((RT, 128), lambda r: (r, 0)),
      out_shape=jax.ShapeDtypeStruct((N, D_OUT), jnp.float32),
  )(t3, gs3, dinv, b3)


@jax.jit
def kernel(x, edge_index, W1, b1, W2, b2, W3, b3):
  x = x.astype(jnp.float32)
  src = edge_index[0]
  dst = edge_index[1]

  cap0 = NS * CW * CK
  cap1 = NS * CWC1 * CK
  def build(v, fill):
    vp = jnp.concatenate([v, jnp.full((cap0 + cap1 - E,), fill, jnp.int32)])
    a = vp[:cap0].reshape(NS, CW, CK)
    b = vp[cap0:].reshape(NS, CWC1, CK)
    b = jnp.concatenate(
        [b, jnp.full((NS, CW - CWC1, CK), fill, jnp.int32)], axis=1)
    return jnp.stack([a, b], axis=1).reshape(NW * CW, CK)
  dstb = build(dst, N)
  pkb = build(src * 16384 + dst, N)
  zero1 = jnp.zeros((NPAD1,), jnp.float32)
  zero2 = jnp.zeros((NPAD2, 128), jnp.float32)
  b1r = b1.reshape(1, D_H1)
  b2r = b2.reshape(8, 1, 128)
  b3r = b3.reshape(1, 128)

  deg_kernel, agg128_kernel, agg1024_kernel = _sc_kernels()
  deg_parts = deg_kernel(dstb, zero1).reshape(NC, NPAD1)
  dinv, xs = _tc_prep(deg_parts, x)

  t1 = agg128_kernel(xs, pkb, zero2)
  h1 = _tc_m1(t1, xs, dinv, W1, b1r)
  w2b = W2.astype(jnp.bfloat16)
  gs2a = _tc_m2(h1, dinv, w2b[:, :512])
  t2a = agg1024_kernel(gs2a, pkb, zero2)
  gs2b = _tc_m2(h1, dinv, w2b[:, 512:])
  t2b = agg1024_kernel(gs2b, pkb, zero2)
  part = _tc_m3a(t2a, gs2a, dinv, b2r[:4], W3[:512])
  gs3 = _tc_m3b(t2b, gs2b, dinv, b2r[4:], W3[512:], part)
  t3 = agg128_kernel(gs3, pkb, zero2)
  return _tc_final(t3, gs3, dinv, b3r)


# speedup vs baseline: 1.5315x; 1.5315x over previous
import functools

import jax
import jax.numpy as jnp
from jax import lax
from jax.experimental import pallas as pl
from jax.experimental.pallas import tpu as pltpu
from jax.experimental.pallas import tpu_sc as plsc

N = 10000
E = 160000
D_IN = 128
D_H1 = 4096
D_H2 = 1024
D_OUT = 128

NC = 2
NS = 16
NW = NC * NS

CK = 64
CW = 120
CWC1 = 37
NPAD2 = 10112
NPAD1 = 10240
RPS = NPAD2 // NS
DPS = NPAD1 // NS

def _worker_id():
  c = lax.axis_index("c")
  s = lax.axis_index("s")
  return s * NC + c


def _deg_body(dstb_hbm, zero1_hbm, out_hbm, dst_v, ones_v, acc_sh):
  c = lax.axis_index("c")
  s = lax.axis_index("s")
  w = _worker_id()

  def fill_ones(i, _):
    ones_v[pl.ds(i * 16, 16)] = jnp.ones((16,), jnp.float32)
    return ()
  lax.fori_loop(0, CK // 16, fill_ones, ())

  pltpu.sync_copy(zero1_hbm.at[pl.ds(s * DPS, DPS)],
                  acc_sh.at[pl.ds(s * DPS, DPS)])
  pltpu.sync_copy(dstb_hbm.at[pl.ds(w * CW, CW)], dst_v)
  plsc.subcore_barrier()

  def chunk(g, _):
    pltpu.sync_copy(ones_v, acc_sh.at[dst_v.at[g]], add=True)
    return ()
  lax.fori_loop(0, jnp.where(c == 0, CW, CWC1), chunk, ())

  plsc.subcore_barrier()
  off = pl.multiple_of(c * NPAD1 + s * DPS, 128)
  pltpu.sync_copy(acc_sh.at[pl.ds(s * DPS, DPS)], out_hbm.at[pl.ds(off, DPS)])


NBUF = 4
LA = 3


def _agg_body(table_hbm, pkb_hbm, zero2_hbm, out_hbm,
              pk_v, sidx_v, didx_v, rows_v, acc_sh, *sems, nplanes, out4d):
  gsems, ssems = sems[:NBUF], sems[NBUF:]
  c = lax.axis_index("c")
  s = lax.axis_index("s")
  w = _worker_id()
  c0 = c == 0

  pltpu.sync_copy(pkb_hbm.at[pl.ds(w * CW, CW)], pk_v)

  def run_pass(table_f, dump_dst):
    pltpu.sync_copy(zero2_hbm.at[pl.ds(s * RPS, RPS)],
                    acc_sh.at[pl.ds(s * RPS, RPS)])
    plsc.subcore_barrier()

    def g_start(g, b):
      for j in range(CK // 16):
        v = pk_v[g, pl.ds(j * 16, 16)]
        sidx_v[b, pl.ds(j * 16, 16)] = lax.shift_right_logical(v, 14)
        didx_v[b, pl.ds(j * 16, 16)] = v & 16383
      pltpu.make_async_copy(
          table_f.at[sidx_v.at[b]], rows_v.at[b], gsems[b]).start()

    def g_wait(b):
      pltpu.make_async_copy(
          table_f.at[sidx_v.at[b]], rows_v.at[b], gsems[b]).wait()

    def s_start(b):
      pltpu.make_async_copy(
          rows_v.at[b], acc_sh.at[didx_v.at[b]], ssems[b]).start(add=True)

    def s_wait(b):
      pltpu.make_async_copy(
          rows_v.at[b], acc_sh.at[didx_v.at[b]], ssems[b]).wait()

    for g in range(LA):
      g_start(g, g % NBUF)
    for g in range(CW):
      b = g % NBUF

      def work(b=b):
        g_wait(b)
        s_start(b)
      if g < CWC1:
        work()
      else:
        pl.when(c0)(work)
      la = g + LA
      if la < CW:
        lb = la % NBUF

        def ahead(la=la, lb=lb):
          if la >= NBUF:
            s_wait(lb)
          g_start(la, lb)
        if la < CWC1:
          ahead()
        else:
          pl.when(c0)(ahead)

    def drain0():
      for g in range(CW - NBUF, CW):
        s_wait(g % NBUF)

    def drain1():
      for g in range(CWC1 - NBUF, CWC1):
        s_wait(g % NBUF)
    pl.when(c0)(drain0)
    pl.when(c != 0)(drain1)

    plsc.subcore_barrier()
    pltpu.sync_copy(acc_sh.at[pl.ds(s * RPS, RPS)], dump_dst)

  if not out4d:
    run_pass(table_hbm, out_hbm.at[c, pl.ds(s * RPS, RPS)])
  else:
    def pass_body(f, _):
      run_pass(table_hbm.at[f], out_hbm.at[c, f, pl.ds(s * RPS, RPS)])
      plsc.subcore_barrier()
      return ()
    lax.fori_loop(0, nplanes, pass_body, ())


@functools.cache
def _sc_kernels():
  mesh = plsc.VectorSubcoreMesh(
      core_axis_name="c", subcore_axis_name="s", num_cores=NC, num_subcores=NS)
  agg_scratch = [
      pltpu.VMEM((CW, CK), jnp.int32),
      pltpu.VMEM((NBUF, CK), jnp.int32),
      pltpu.VMEM((NBUF, CK), jnp.int32),
      pltpu.VMEM((NBUF, CK, 128), jnp.float32),
      pltpu.VMEM_SHARED((NPAD2, 128), jnp.float32),
  ] + [pltpu.SemaphoreType.DMA] * (2 * NBUF)

  deg = pl.kernel(
      _deg_body,
      out_type=jax.ShapeDtypeStruct((NC * NPAD1,), jnp.float32),
      mesh=mesh,
      scratch_types=[
          pltpu.VMEM((CW, CK), jnp.int32),
          pltpu.VMEM((CK,), jnp.float32),
          pltpu.VMEM_SHARED((NPAD1,), jnp.float32),
      ],
  )

  def agg128_body(table_hbm, pkb_hbm, zero2_hbm, out_hbm,
                  pk_v, sidx_v, didx_v, rows_v, acc_sh, *sems):
    _agg_body(table_hbm, pkb_hbm, zero2_hbm, out_hbm,
              pk_v, sidx_v, didx_v, rows_v, acc_sh, *sems,
              nplanes=1, out4d=False)

  def agg1024_body(table_hbm, pkb_hbm, zero2_hbm, out_hbm,
                   pk_v, sidx_v, didx_v, rows_v, acc_sh, *sems):
    _agg_body(table_hbm, pkb_hbm, zero2_hbm, out_hbm,
              pk_v, sidx_v, didx_v, rows_v, acc_sh, *sems,
              nplanes=4, out4d=True)

  agg128 = pl.kernel(
      agg128_body,
      out_type=jax.ShapeDtypeStruct((NC, NPAD2, 128), jnp.float32),
      mesh=mesh, scratch_types=agg_scratch)
  agg1024 = pl.kernel(
      agg1024_body,
      out_type=jax.ShapeDtypeStruct((NC, 4, NPAD2, 128), jnp.float32),
      mesh=mesh, scratch_types=agg_scratch)
  return deg, agg128, agg1024


RT = 400
MT = 2000
NGRID = N // RT


def _prep_body(deg_ref, x_ref, dinv_ref, xs_ref):
  ones = jnp.ones((NC, 128), jnp.float32)
  degb = lax.dot_general(deg_ref[...], ones, (((0,), (0,)), ((), ())),
                         preferred_element_type=jnp.float32)
  dinv = lax.rsqrt(degb + 1.0)
  dinv_ref[...] = dinv
  xs_ref[...] = x_ref[...] * dinv


def _tc_prep(deg_parts, x):
  rp = 512
  return pl.pallas_call(
      _prep_body,
      grid=(pl.cdiv(N, rp),),
      in_specs=[
          pl.BlockSpec((NC, rp), lambda r: (0, r)),
          pl.BlockSpec((rp, 128), lambda r: (r, 0)),
      ],
      out_specs=[
          pl.BlockSpec((rp, 128), lambda r: (r, 0)),
          pl.BlockSpec((rp, 128), lambda r: (r, 0)),
      ],
      out_shape=[
          jax.ShapeDtypeStruct((N, 128), jnp.float32),
          jax.ShapeDtypeStruct((N, 128), jnp.float32),
      ],
  )(deg_parts, x)


def _m1_body(t1_ref, xs_ref, dinv_ref, w_ref, b_ref, out_ref):
  u = dinv_ref[...] * (t1_ref[0] + t1_ref[1] + xs_ref[...])
  for n in range(D_H1 // 512):
    a = jnp.dot(u, w_ref[:, n * 512:(n + 1) * 512],
                preferred_element_type=jnp.float32)
    out_ref[:, n * 512:(n + 1) * 512] = jnp.maximum(
        a + b_ref[:, n * 512:(n + 1) * 512], 0.0).astype(jnp.bfloat16)


def _tc_m1(t1, xs, dinv, w1, b1):
  return pl.pallas_call(
      _m1_body,
      grid=(N // MT,),
      in_specs=[
          pl.BlockSpec((NC, MT, 128), lambda r: (0, r, 0)),
          pl.BlockSpec((MT, 128), lambda r: (r, 0)),
          pl.BlockSpec((MT, 128), lambda r: (r, 0)),
          pl.BlockSpec((128, D_H1), lambda r: (0, 0)),
          pl.BlockSpec((1, D_H1), lambda r: (0, 0)),
      ],
      out_specs=pl.BlockSpec((MT, D_H1), lambda r: (r, 0)),
      out_shape=jax.ShapeDtypeStruct((N, D_H1), jnp.bfloat16),
  )(t1, xs, dinv, w1, b1)


def _m2_body(h1_ref, w_ref, dinv_ref, out_ref, acc_ref):
  k = pl.program_id(1)

  @pl.when(k == 0)
  def _():
    acc_ref[...] = jnp.zeros_like(acc_ref)

  acc_ref[...] += jnp.dot(h1_ref[...], w_ref[...],
                          preferred_element_type=jnp.float32)

  @pl.when(k == pl.num_programs(1) - 1)
  def _():
    dinv = dinv_ref[...]
    for t in range(4):
      out_ref[t] = dinv * acc_ref[:, t * 128:(t + 1) * 128]


def _tc_m2(h1, dinv, w2half):
  kk = 512
  return pl.pallas_call(
      _m2_body,
      grid=(N // MT, D_H1 // kk),
      in_specs=[
          pl.BlockSpec((MT, kk), lambda r, k: (r, k)),
          pl.BlockSpec((kk, 512), lambda r, k: (k, 0)),
          pl.BlockSpec((MT, 128), lambda r, k: (r, 0)),
      ],
      out_specs=pl.BlockSpec((4, MT, 128), lambda r, k: (0, r, 0)),
      out_shape=jax.ShapeDtypeStruct((4, N, 128), jnp.float32),
      scratch_shapes=[pltpu.VMEM((MT, 512), jnp.float32)],
      compiler_params=pltpu.CompilerParams(
          dimension_semantics=("parallel", "arbitrary")),
  )(h1, w2half, dinv)


def _m3a_body(t2_ref, gs2_ref, dinv_ref, b2_ref, w3_ref, out_ref, acc_ref):
  dinv = dinv_ref[...]
  acc_ref[...] = jnp.zeros_like(acc_ref)
  for f in range(4):
    h2 = jnp.maximum(
        dinv * (t2_ref[0, f] + t2_ref[1, f] + gs2_ref[f]) + b2_ref[f], 0.0)
    acc_ref[...] += jnp.dot(h2, w3_ref[f * 128:(f + 1) * 128],
                            preferred_element_type=jnp.float32)
  out_ref[...] = acc_ref[...]


def _m3b_body(t2_ref, gs2_ref, dinv_ref, b2_ref, w3_ref, part_ref,
              out_ref, acc_ref):
  dinv = dinv_ref[...]
  acc_ref[...] = part_ref[...]
  for f in range(4):
    h2 = jnp.maximum(
        dinv * (t2_ref[0, f] + t2_ref[1, f] + gs2_ref[f]) + b2_ref[f], 0.0)
    acc_ref[...] += jnp.dot(h2, w3_ref[f * 128:(f + 1) * 128],
                            preferred_element_type=jnp.float32)
  out_ref[...] = dinv * acc_ref[...]


def _m3_specs(extra):
  return dict(
      grid=(N // MT,),
      in_specs=[
          pl.BlockSpec((NC, 4, MT, 128), lambda r: (0, 0, r, 0)),
          pl.BlockSpec((4, MT, 128), lambda r: (0, r, 0)),
          pl.BlockSpec((MT, 128), lambda r: (r, 0)),
          pl.BlockSpec((4, 1, 128), lambda r: (0, 0, 0)),
          pl.BlockSpec((512, 128), lambda r: (0, 0)),
      ] + extra,
      out_specs=pl.BlockSpec((MT, 128), lambda r: (r, 0)),
      out_shape=jax.ShapeDtypeStruct((N, 128), jnp.float32),
      scratch_shapes=[pltpu.VMEM((MT, 128), jnp.float32)],
  )


def _tc_m3a(t2, gs2, dinv, b2, w3):
  return pl.pallas_call(_m3a_body, **_m3_specs([]))(t2, gs2, dinv, b2, w3)


def _tc_m3b(t2, gs2, dinv, b2, w3, part):
  return pl.pallas_call(
      _m3b_body,
      **_m3_specs([pl.BlockSpec((MT, 128), lambda r: (r, 0))]),
  )(t2, gs2, dinv, b2, w3, part)


def _final_body(t3_ref, gs3_ref, dinv_ref, b3_ref, out_ref):
  u = dinv_ref[...] * (t3_ref[0] + t3_ref[1] + gs3_ref[...]) + b3_ref[...]
  m = jnp.max(u, axis=1, keepdims=True)
  e = jnp.exp(u - m)
  lse = jnp.log(jnp.sum(e, axis=1, keepdims=True)) + m
  out_ref[...] = u - lse


def _tc_final(t3, gs3, dinv, b3):
  return pl.pallas_call(
      _final_body,
      grid=(NGRID,),
      in_specs=[
          pl.BlockSpec((NC, RT, 128), lambda r: (0, r, 0)),
          pl.BlockSpec((RT, 128), lambda r: (r, 0)),
          pl.BlockSpec((RT, 128), lambda r: (r, 0)),
          pl.BlockSpec((1, 128), lambda r: (0, 0)),
      ],
      out_specs=pl.BlockSpec((RT, 128), lambda r: (r, 0)),
      out_shape=jax.ShapeDtypeStruct((N, D_OUT), jnp.float32),
  )(t3, gs3, dinv, b3)


@jax.jit
def kernel(x, edge_index, W1, b1, W2, b2, W3, b3):
  x = x.astype(jnp.float32)
  src = edge_index[0]
  dst = edge_index[1]

  cap0 = NS * CW * CK
  cap1 = NS * CWC1 * CK
  def build(v, fill):
    vp = jnp.concatenate([v, jnp.full((cap0 + cap1 - E,), fill, jnp.int32)])
    a = vp[:cap0].reshape(NS, CW, CK)
    b = vp[cap0:].reshape(NS, CWC1, CK)
    b = jnp.concatenate(
        [b, jnp.full((NS, CW - CWC1, CK), fill, jnp.int32)], axis=1)
    return jnp.stack([a, b], axis=1).reshape(NW * CW, CK)
  dstb = build(dst, N)
  pkb = build(src * 16384 + dst, N)
  zero1 = jnp.zeros((NPAD1,), jnp.float32)
  zero2 = jnp.zeros((NPAD2, 128), jnp.float32)
  b1r = b1.reshape(1, D_H1)
  b2r = b2.reshape(8, 1, 128)
  b3r = b3.reshape(1, 128)

  deg_kernel, agg128_kernel, agg1024_kernel = _sc_kernels()
  deg_parts = deg_kernel(dstb, zero1).reshape(NC, NPAD1)
  dinv, xs = _tc_prep(deg_parts, x)

  t1 = agg128_kernel(xs, pkb, zero2)
  h1 = _tc_m1(t1, xs, dinv, W1, b1r)
  w2b = W2.astype(jnp.bfloat16)
  gs2a = _tc_m2(h1, dinv, w2b[:, :512])
  t2a = agg1024_kernel(gs2a, pkb, zero2)
  gs2b = _tc_m2(h1, dinv, w2b[:, 512:])
  t2b = agg1024_kernel(gs2b, pkb, zero2)
  part = _tc_m3a(t2a, gs2a, dinv, b2r[:4], W3[:512])
  gs3 = _tc_m3b(t2b, gs2b, dinv, b2r[4:], W3[512:], part)
  t3 = agg128_kernel(gs3, pkb, zero2)
  return _tc_final(t3, gs3, dinv, b3r)
